# baseline (speedup 1.0000x reference)
import functools

import jax
import jax.numpy as jnp
from jax import lax
from jax.experimental import pallas as pl
from jax.experimental.pallas import tpu as pltpu
from jax.experimental.pallas import tpu_sc as plsc

NC = 2
NS = 16
NW = NC * NS
LANES = 16


NBUF = 3
UNROLL = 2
ACC_ROWS = 16


def _make_pool(vocab, embed, batch, seq_chunks, chunk):
  rows_per_w = batch // NW
  nreg = embed // LANES
  mesh = plsc.VectorSubcoreMesh(
      core_axis_name="c", subcore_axis_name="s",
      num_cores=NC, num_subcores=NS)

  def body(x_hbm, emb_hbm, out_hbm, idx_v, buf_v, acc_v, sems):
    wid = lax.axis_index("s") * NC + lax.axis_index("c")
    base = wid * rows_per_w
    pltpu.sync_copy(x_hbm.at[pl.ds(base, rows_per_w)], idx_v)

    def fire_chunk(b, p, j):
      pltpu.async_copy(emb_hbm.at[idx_v.at[b, j]], buf_v.at[p, j],
                       sems.at[p, j])

    def fire(b, p):
      for j in range(seq_chunks):
        fire_chunk(b, p, j)

    def wait(p, j):
      pltpu.make_async_copy(
          emb_hbm.at[idx_v.at[0, j]], buf_v.at[p, j], sems.at[p, j]).wait()

    lax.fori_loop(0, NBUF, lambda i, c: (fire(i, i), c)[1], 0)

    def accum(p, b):
      def tok(j):
        def f(t, acc):
          for u in range(UNROLL):
            acc = tuple(
                acc[k] + buf_v[p, j, UNROLL * t + u, pl.ds(LANES * k, LANES)]
                for k in range(nreg))
          return acc
        return f
      acc = tuple(jnp.zeros((LANES,), jnp.float32) for _ in range(nreg))
      for j in range(seq_chunks):
        wait(p, j)
        acc = lax.fori_loop(0, chunk // UNROLL, tok(j), acc)

        @pl.when(b + NBUF < rows_per_w)
        def _():
          fire_chunk(b + NBUF, p, j)
      for k in range(nreg):
        acc_v[b % ACC_ROWS, pl.ds(LANES * k, LANES)] = acc[k]

    def step(b, carry):
      p = b % NBUF
      accum(p, b)

      @pl.when((b + 1) % ACC_ROWS == 0)
      def _():
        flush_base = pl.multiple_of(base + b + 1 - ACC_ROWS, ACC_ROWS)
        pltpu.sync_copy(acc_v, out_hbm.at[pl.ds(flush_base, ACC_ROWS)])
      return carry

    lax.fori_loop(0, rows_per_w, step, 0)

  return pl.kernel(
      body,
      out_type=jax.ShapeDtypeStruct((batch, embed), jnp.float32),
      mesh=mesh,
      scratch_types=[
          pltpu.VMEM((rows_per_w, seq_chunks, chunk), jnp.int32),
          pltpu.VMEM((NBUF, seq_chunks, chunk, embed), jnp.float32),
          pltpu.VMEM((ACC_ROWS, embed), jnp.float32),
          pltpu.SemaphoreType.DMA((NBUF, seq_chunks)),
      ],
  )


def _mlp_body(inv_l, p_ref, w1_ref, b1_ref, w2_ref, b2_ref, o_ref):
  pooled = p_ref[:] * inv_l
  h = jnp.maximum(
      jnp.dot(pooled, w1_ref[:], preferred_element_type=jnp.float32)
      + b1_ref[:][None, :], 0.0)
  out_t = lax.dot_general(
      w2_ref[:], h, (((0,), (1,)), ((), ())),
      preferred_element_type=jnp.float32)
  o_ref[:] = out_t + b2_ref[:][:, None]


def kernel(x, embedding, W1, b1, W2, b2):
  batch, seq = x.shape
  vocab, embed = embedding.shape
  hidden = W1.shape[1]
  ncls = W2.shape[1]
  del hidden
  chunk = 100
  seq_chunks = seq // chunk

  xr = x.astype(jnp.int32).reshape(batch, seq_chunks, chunk)
  pool = _make_pool(vocab, embed, batch, seq_chunks, chunk)
  pooled_sum = pool(xr, embedding)

  mlp = pl.pallas_call(
      functools.partial(_mlp_body, 1.0 / seq),
      out_shape=jax.ShapeDtypeStruct((ncls, batch), jnp.float32),
  )
  return mlp(pooled_sum, W1, b1, W2, b2).T


# speedup vs baseline: 1.0067x; 1.0067x over previous
import functools

import jax
import jax.numpy as jnp
from jax import lax
from jax.experimental import pallas as pl
from jax.experimental.pallas import tpu as pltpu
from jax.experimental.pallas import tpu_sc as plsc

NC = 2
NS = 16
NW = NC * NS
LANES = 16


NBUF = 3
UNROLL = 2
ACC_ROWS = 16


def _make_pool(vocab, embed, batch, seq_chunks, chunk):
  rows_per_w = batch // NW
  nreg = embed // LANES
  mesh = plsc.VectorSubcoreMesh(
      core_axis_name="c", subcore_axis_name="s",
      num_cores=NC, num_subcores=NS)

  def body(x_hbm, emb_hbm, out_hbm, idx_v, buf_v, acc_v, sems):
    wid = lax.axis_index("s") * NC + lax.axis_index("c")
    base = wid * rows_per_w
    pltpu.sync_copy(x_hbm.at[pl.ds(base, rows_per_w)], idx_v)

    def fire_chunk(b, p, j):
      pltpu.async_copy(emb_hbm.at[idx_v.at[b, j]], buf_v.at[p, j],
                       sems.at[p, j])

    def fire(b, p):
      for j in range(seq_chunks):
        fire_chunk(b, p, j)

    def wait(p, j):
      pltpu.make_async_copy(
          emb_hbm.at[idx_v.at[0, j]], buf_v.at[p, j], sems.at[p, j]).wait()

    lax.fori_loop(0, NBUF, lambda i, c: (fire(i, i), c)[1], 0)

    def accum(p, b):
      def tok(j):
        def f(t, acc):
          for u in range(UNROLL):
            acc = tuple(
                acc[k] + buf_v[p, j, UNROLL * t + u, pl.ds(LANES * k, LANES)]
                for k in range(nreg))
          return acc
        return f
      acc = tuple(jnp.zeros((LANES,), jnp.float32) for _ in range(nreg))
      for j in range(seq_chunks):
        wait(p, j)
        acc = lax.fori_loop(0, chunk // UNROLL, tok(j), acc)
      for k in range(nreg):
        acc_v[b % ACC_ROWS, pl.ds(LANES * k, LANES)] = acc[k]

    def step(b, carry):
      p = b % NBUF
      accum(p, b)

      @pl.when(b + NBUF < rows_per_w)
      def _():
        fire(b + NBUF, p)

      @pl.when((b + 1) % ACC_ROWS == 0)
      def _():
        flush_base = pl.multiple_of(base + b + 1 - ACC_ROWS, ACC_ROWS)
        pltpu.sync_copy(acc_v, out_hbm.at[pl.ds(flush_base, ACC_ROWS)])
      return carry

    lax.fori_loop(0, rows_per_w, step, 0)

  return pl.kernel(
      body,
      out_type=jax.ShapeDtypeStruct((batch, embed), jnp.float32),
      mesh=mesh,
      scratch_types=[
          pltpu.VMEM((rows_per_w, seq_chunks, chunk), jnp.int32),
          pltpu.VMEM((NBUF, seq_chunks, chunk, embed), jnp.float32),
          pltpu.VMEM((ACC_ROWS, embed), jnp.float32),
          pltpu.SemaphoreType.DMA((NBUF, seq_chunks)),
      ],
  )


def _mlp_body(inv_l, p_ref, w1_ref, b1_ref, w2_ref, b2_ref, o_ref):
  pooled = p_ref[:] * inv_l
  h = jnp.maximum(
      jnp.dot(pooled, w1_ref[:], preferred_element_type=jnp.float32)
      + b1_ref[:][None, :], 0.0)
  out_t = lax.dot_general(
      w2_ref[:], h, (((0,), (1,)), ((), ())),
      preferred_element_type=jnp.float32)
  o_ref[:] = out_t + b2_ref[:][:, None]


def kernel(x, embedding, W1, b1, W2, b2):
  batch, seq = x.shape
  vocab, embed = embedding.shape
  hidden = W1.shape[1]
  ncls = W2.shape[1]
  del hidden
  chunk = 100
  seq_chunks = seq // chunk

  xr = x.astype(jnp.int32).reshape(batch, seq_chunks, chunk)
  pool = _make_pool(vocab, embed, batch, seq_chunks, chunk)
  pooled_sum = pool(xr, embedding)

  mlp = pl.pallas_call(
      functools.partial(_mlp_body, 1.0 / seq),
      out_shape=jax.ShapeDtypeStruct((ncls, batch), jnp.float32),
  )
  return mlp(pooled_sum, W1, b1, W2, b2).T
